# baseline (speedup 1.0000x reference)
import jax
import jax.numpy as jnp
from jax import lax
from jax.experimental import pallas as pl
from jax.experimental.pallas import tpu as pltpu
from jax.experimental.pallas import tpu_sc as plsc

_N_VOCAB = 1000000
_N_EMBED = 64
_BATCH = 16384
_A_ROWS = 600000

_info = plsc.get_sparse_core_info()
_NC = _info.num_cores
_NS = _info.num_subcores
_NW = _NC * _NS
_B_PER_W = _BATCH // _NW
_K = 16
_NBATCH = _B_PER_W // _K


def _gather_kernel(tbl_a, tbl_b, idx_hbm, out_hbm, idx_v, rows_v, sem):
    wid = lax.axis_index("s") * _NC + lax.axis_index("c")
    base = wid * _B_PER_W
    pltpu.sync_copy(idx_hbm.at[wid], idx_v)

    def batch_body(b, _):
        vblk = idx_v[pl.ds(b * _K, _K)]
        for l in range(_K):
            i = vblk[l]
            j = b * _K + l

            @pl.when(i < _A_ROWS)
            def _():
                pltpu.async_copy(
                    tbl_a.at[0, pl.ds(i, 1), :],
                    rows_v.at[pl.ds(j, 1), :],
                    sem,
                )

            @pl.when(i >= _A_ROWS)
            def _():
                pltpu.async_copy(
                    tbl_b.at[pl.ds(i - _A_ROWS, 1), :],
                    rows_v.at[pl.ds(j, 1), :],
                    sem,
                )
        return ()

    lax.fori_loop(0, _NBATCH, batch_body, (), unroll=False)

    def drain_body(b, _):
        pltpu.make_async_copy(
            tbl_b.at[pl.ds(0, 1), :], rows_v.at[pl.ds(0, 1), :], sem
        ).wait()
        return ()

    lax.fori_loop(0, _B_PER_W, drain_body, (), unroll=False)
    pltpu.sync_copy(rows_v, out_hbm.at[pl.ds(base, _B_PER_W)])


@jax.jit
def kernel(input_words, in_embed_weight):
    idx = input_words.reshape(_NW, _B_PER_W)
    tbl_a = in_embed_weight[:_A_ROWS].reshape(1, _A_ROWS, _N_EMBED)
    tbl_b = in_embed_weight[_A_ROWS:]
    mesh = plsc.VectorSubcoreMesh(core_axis_name="c", subcore_axis_name="s")
    out = pl.kernel(
        _gather_kernel,
        mesh=mesh,
        out_type=jax.ShapeDtypeStruct((_BATCH, _N_EMBED), jnp.float32),
        scratch_types=[
            pltpu.VMEM((_B_PER_W,), jnp.int32),
            pltpu.VMEM((_B_PER_W, _N_EMBED), jnp.float32),
            pltpu.SemaphoreType.DMA,
        ],
    )(tbl_a, tbl_b, idx)
    return out


# speedup vs baseline: 1.7197x; 1.7197x over previous
import functools

import jax
import jax.numpy as jnp
from jax import lax
from jax.experimental import pallas as pl
from jax.experimental.pallas import tpu as pltpu
from jax.experimental.pallas import tpu_sc as plsc

_N_VOCAB = 1000000
_N_EMBED = 64
_BATCH = 16384

_info = plsc.get_sparse_core_info()
_NC = _info.num_cores
_NS = _info.num_subcores
_NW = _NC * _NS
_B_PER_W = _BATCH // _NW
_K = 16
_NBATCH = _B_PER_W // _K


def _gather_kernel(tbl_hbm, idx_hbm, out_hbm, idx_v, rows_v, sem):
    wid = lax.axis_index("s") * _NC + lax.axis_index("c")
    base = wid * _B_PER_W
    pltpu.sync_copy(idx_hbm.at[wid], idx_v)

    def batch_body(b, _):
        vblk = idx_v[pl.ds(b * _K, _K)]
        for l in range(_K):
            i = vblk[l]
            pltpu.async_copy(
                tbl_hbm.at[0, pl.ds(i, 1), :],
                rows_v.at[pl.ds(b * _K + l, 1), :],
                sem,
            )
        return ()

    lax.fori_loop(0, _NBATCH, batch_body, (), unroll=False)

    def drain_body(b, _):
        pltpu.make_async_copy(
            tbl_hbm.at[0, pl.ds(0, 1), :], rows_v.at[pl.ds(0, 1), :], sem
        ).wait()
        return ()

    lax.fori_loop(0, _B_PER_W, drain_body, (), unroll=False)
    pltpu.sync_copy(rows_v, out_hbm.at[pl.ds(base, _B_PER_W)])


@jax.jit
def kernel(input_words, in_embed_weight):
    idx = input_words.reshape(_NW, _B_PER_W)
    tbl = in_embed_weight.reshape(1, _N_VOCAB, _N_EMBED)
    mesh = plsc.VectorSubcoreMesh(core_axis_name="c", subcore_axis_name="s")
    out = pl.kernel(
        _gather_kernel,
        mesh=mesh,
        out_type=jax.ShapeDtypeStruct((_BATCH, _N_EMBED), jnp.float32),
        scratch_types=[
            pltpu.VMEM((_B_PER_W,), jnp.int32),
            pltpu.VMEM((_B_PER_W, _N_EMBED), jnp.float32),
            pltpu.SemaphoreType.DMA,
        ],
    )(tbl, idx)
    return out
